# baseline (speedup 1.0000x reference)
import functools

import jax
import jax.numpy as jnp
from jax import lax
from jax.experimental import pallas as pl
from jax.experimental.pallas import tpu as pltpu
from jax.experimental.pallas import tpu_sc as plsc

Q = 1024
D = 64
K = 100000
CHUNK = 2048
NCHUNK = 49
KPAD = NCHUNK * CHUNK
GROUP = 128
GPC = CHUNK // GROUP
NGROUP = NCHUNK * GPC
NEG = -3.0e38

_NTILE = 32
_QPT = Q // _NTILE


_JPART = (K % CHUNK) // GROUP


def _score_kernel(feat_ref, keys_ref, s_ref, gmax_ref, qn_scr):
    i = pl.program_id(0)

    @pl.when(i == 0)
    def _():
        f = feat_ref[...]
        qn_scr[...] = f * lax.rsqrt(jnp.sum(f * f, axis=1, keepdims=True))

    qn = qn_scr[...]
    k = keys_ref[...]
    kn = k * lax.rsqrt(jnp.maximum(jnp.sum(k * k, axis=1, keepdims=True), 1e-30))
    last = i == NCHUNK - 1
    parts = []
    gparts = []
    for j in range(GPC):
        knj = kn[j * GROUP:(j + 1) * GROUP]
        sj = lax.dot_general(qn, knj, (((1,), (1,)), ((), ())),
                             preferred_element_type=jnp.float32)
        if j == _JPART:
            col = lax.broadcasted_iota(jnp.int32, (Q, GROUP), 1) \
                + i * CHUNK + j * GROUP
            sj = jnp.where(col < K, sj, NEG)
        gj = jnp.max(sj, axis=1, keepdims=True)
        if j > _JPART:
            gj = jnp.where(last, jnp.full((Q, 1), NEG, jnp.float32), gj)
        parts.append(sj)
        gparts.append(gj)
    s_ref[...] = jnp.stack(parts, axis=0)
    g16 = jnp.concatenate(gparts, axis=1)
    gmax_ref[0] = jnp.concatenate(
        [g16, jnp.full((Q, GROUP - GPC), NEG, jnp.float32)], axis=1)


def _merge16(run_asc, new_desc):
    return jnp.maximum(run_asc, new_desc)


def _select_body(sg_hbm, gmax_hbm, out_hbm, gm_v, rows_v, res_v, sem):
    c = lax.axis_index("c")
    s_ = lax.axis_index("s")
    wid = s_ * 2 + c
    q0 = wid * _QPT
    pltpu.sync_copy(gmax_hbm.at[:, pl.ds(q0, _QPT), pl.ds(0, GPC)], gm_v)

    def phase2(qi, _):
        qg = q0 + qi

        def chunk_step(cc, carry):
            rv, ri = carry
            g = gm_v[cc, qi, :]

            def merge(rv, ri):
                ids = lax.iota(jnp.int32, 16) + cc * 16
                gs_n, gis = lax.sort((-g, ids), dimension=0, num_keys=1)
                gs = -gs_n
                take = rv >= gs
                mv = jnp.where(take, rv, gs)
                mi = jnp.where(take, ri, gis)
                rv2, ri2 = lax.sort((mv, mi), dimension=0, num_keys=1)
                return (rv2, ri2)

            return lax.cond(jnp.max(g) < rv[0],
                            lambda rv, ri: (rv, ri), merge, rv, ri)

        rv0 = jnp.full((16,), NEG, jnp.float32)
        ri0 = jnp.zeros((16,), jnp.int32)
        rv, ri = lax.fori_loop(0, NCHUNK, chunk_step, (rv0, ri0))
        for j in range(16):
            pltpu.async_copy(sg_hbm.at[ri[15 - j], qg], rows_v.at[qi, j], sem)
        return 0

    lax.fori_loop(0, _QPT, phase2, 0)

    def drain(qi, _):
        pltpu.make_async_copy(
            sg_hbm.at[0, pl.ds(0, 16), :], rows_v.at[qi], sem).wait()
        return 0

    lax.fori_loop(0, _QPT, drain, 0)

    def phase3(qi, accs):
        def grp(gi, run):
            def vstep(j, run2):
                v = rows_v[qi, gi, pl.ds(j * 16, 16)]

                def merge(run2):
                    vd = lax.rev(lax.sort(v, dimension=0), (0,))
                    return lax.sort(_merge16(run2, vd), dimension=0)

                return lax.cond(jnp.max(v) < run2[6],
                                lambda run2: run2, merge, run2)
            return lax.fori_loop(0, GROUP // 16, vstep, run)

        run0 = jnp.full((16,), NEG, jnp.float32)
        run = lax.fori_loop(0, 16, grp, run0)
        r = 2.0 - 2.0 * run[6]
        acc0, acc1 = accs
        lane = lax.iota(jnp.int32, 16)
        acc0 = jnp.where((qi < 16) & (lane == qi), r, acc0)
        acc1 = jnp.where((qi >= 16) & (lane == qi - 16), r, acc1)
        return (acc0, acc1)

    z = jnp.zeros((16,), jnp.float32)
    acc0, acc1 = lax.fori_loop(0, _QPT, phase3, (z, z))
    res_v[pl.ds(0, 16)] = acc0
    res_v[pl.ds(16, 16)] = acc1
    pltpu.sync_copy(res_v, out_hbm.at[pl.ds(q0, _QPT)])


@functools.cache
def _get_select():
    @functools.partial(
        pl.kernel,
        out_type=jax.ShapeDtypeStruct((Q,), jnp.float32),
        mesh=plsc.VectorSubcoreMesh(core_axis_name="c", subcore_axis_name="s"),
        scratch_types=[
            pltpu.VMEM((NCHUNK, _QPT, 16), jnp.float32),
            pltpu.VMEM((_QPT, 16, GROUP), jnp.float32),
            pltpu.VMEM((_QPT,), jnp.float32),
            pltpu.SemaphoreType.DMA,
        ],
        compiler_params=pltpu.CompilerParams(
            needs_layout_passes=False, use_tc_tiling_on_sc=False
        ),
    )
    def _select(sg_hbm, gmax_hbm, out_hbm, gm_v, rows_v, res_v, sem):
        _select_body(sg_hbm, gmax_hbm, out_hbm, gm_v, rows_v, res_v, sem)

    return _select


def _scores(features, keys_p, interpret=False):
    return pl.pallas_call(
        _score_kernel,
        grid=(NCHUNK,),
        in_specs=[
            pl.BlockSpec((Q, D), lambda i: (0, 0)),
            pl.BlockSpec((CHUNK, D), lambda i: (i, 0)),
        ],
        out_specs=[
            pl.BlockSpec((GPC, Q, GROUP), lambda i: (i, 0, 0)),
            pl.BlockSpec((1, Q, GROUP), lambda i: (i, 0, 0)),
        ],
        out_shape=[
            jax.ShapeDtypeStruct((NGROUP, Q, GROUP), jnp.float32),
            jax.ShapeDtypeStruct((NCHUNK, Q, GROUP), jnp.float32),
        ],
        scratch_shapes=[pltpu.VMEM((Q, D), jnp.float32)],
        interpret=interpret,
    )(features, keys_p)


def kernel(features, logits, keys):
    del logits
    s3, gmax = _scores(features, keys)
    kth = _get_select()(s3, gmax)
    return kth.reshape(Q, 1)


# speedup vs baseline: 1.6545x; 1.6545x over previous
import functools

import jax
import jax.numpy as jnp
from jax import lax
from jax.experimental import pallas as pl
from jax.experimental.pallas import tpu as pltpu
from jax.experimental.pallas import tpu_sc as plsc

Q = 1024
D = 64
K = 100000
CHUNK = 2048
NCHUNK = 49
KPAD = NCHUNK * CHUNK
GROUP = 128
GPC = CHUNK // GROUP
NGROUP = NCHUNK * GPC
NEG = -3.0e38

_NTILE = 32
_QPT = Q // _NTILE


_JPART = (K % CHUNK) // GROUP


def _score_kernel(feat_ref, keys_ref, s_ref, gmax_ref, qn_scr):
    i = pl.program_id(0)

    @pl.when(i == 0)
    def _():
        f = feat_ref[...]
        qn_scr[...] = f * lax.rsqrt(jnp.sum(f * f, axis=1, keepdims=True))

    qn = qn_scr[...]
    k = keys_ref[...]
    kn = k * lax.rsqrt(jnp.maximum(jnp.sum(k * k, axis=1, keepdims=True), 1e-30))
    last = i == NCHUNK - 1
    parts = []
    gparts = []
    for j in range(GPC):
        knj = kn[j * GROUP:(j + 1) * GROUP]
        sj = lax.dot_general(qn, knj, (((1,), (1,)), ((), ())),
                             preferred_element_type=jnp.float32)
        if j == _JPART:
            col = lax.broadcasted_iota(jnp.int32, (Q, GROUP), 1) \
                + i * CHUNK + j * GROUP
            sj = jnp.where(col < K, sj, NEG)
        gj = jnp.max(sj, axis=1, keepdims=True)
        if j > _JPART:
            gj = jnp.where(last, jnp.full((Q, 1), NEG, jnp.float32), gj)
        parts.append(sj)
        gparts.append(gj)
    s_ref[...] = jnp.stack(parts, axis=0)
    g16 = jnp.concatenate(gparts, axis=1)
    gmax_ref[0] = jnp.concatenate(
        [g16, jnp.full((Q, GROUP - GPC), NEG, jnp.float32)], axis=1)


def _merge16(run_asc, new_desc):
    return jnp.maximum(run_asc, new_desc)


def _select_body(sg_hbm, gmax_hbm, out_hbm, gm_v, rows_v, res_v, sem):
    c = lax.axis_index("c")
    s_ = lax.axis_index("s")
    wid = s_ * 2 + c
    q0 = wid * _QPT
    pltpu.sync_copy(gmax_hbm.at[:, pl.ds(q0, _QPT), pl.ds(0, GPC)], gm_v)

    _HQ = _QPT // 2

    def _p2_one(cc, g, rv, ri):
        ids = lax.iota(jnp.int32, 16) + cc * 16
        gs_n, gis = lax.sort((-g, ids), dimension=0, num_keys=1)
        gs = -gs_n
        take = rv >= gs
        mv = jnp.where(take, rv, gs)
        mi = jnp.where(take, ri, gis)
        return lax.sort((mv, mi), dimension=0, num_keys=1)

    def phase2(qi, _):
        def chunk_step(cc, carry):
            rva, ria, rvb, rib = carry
            rva, ria = _p2_one(cc, gm_v[cc, qi, :], rva, ria)
            rvb, rib = _p2_one(cc, gm_v[cc, qi + _HQ, :], rvb, rib)
            return (rva, ria, rvb, rib)

        rv0 = jnp.full((16,), NEG, jnp.float32)
        ri0 = jnp.zeros((16,), jnp.int32)
        rva, ria, rvb, rib = lax.fori_loop(
            0, NCHUNK, chunk_step, (rv0, ri0, rv0, ri0))
        for j in range(6, 16):
            pltpu.async_copy(sg_hbm.at[ria[j], q0 + qi],
                             rows_v.at[qi, j - 6], sem)
            pltpu.async_copy(sg_hbm.at[rib[j], q0 + qi + _HQ],
                             rows_v.at[qi + _HQ, j - 6], sem)
        return 0

    lax.fori_loop(0, _HQ, phase2, 0)

    def drain(qi, _):
        pltpu.make_async_copy(
            sg_hbm.at[0, pl.ds(0, 10), :], rows_v.at[qi], sem).wait()
        return 0

    lax.fori_loop(0, _QPT, drain, 0)

    def _p3_one(v, run2):
        vd = lax.rev(lax.sort(v, dimension=0), (0,))
        return lax.sort(_merge16(run2, vd), dimension=0)

    def phase3(qi, accs):
        def grp(gi, runs):
            def vstep(j, runs2):
                ra, rb = runs2
                ra = _p3_one(rows_v[qi, gi, pl.ds(j * 16, 16)], ra)
                rb = _p3_one(rows_v[qi + _HQ, gi, pl.ds(j * 16, 16)], rb)
                return (ra, rb)
            return lax.fori_loop(0, GROUP // 16, vstep, runs)

        run0 = jnp.full((16,), NEG, jnp.float32)
        ra, rb = lax.fori_loop(0, 10, grp, (run0, run0))
        acc0, acc1 = accs
        lane = lax.iota(jnp.int32, 16)
        acc0 = jnp.where(lane == qi, 2.0 - 2.0 * ra[6], acc0)
        acc1 = jnp.where(lane == qi, 2.0 - 2.0 * rb[6], acc1)
        return (acc0, acc1)

    z = jnp.zeros((16,), jnp.float32)
    acc0, acc1 = lax.fori_loop(0, _HQ, phase3, (z, z))
    res_v[pl.ds(0, 16)] = acc0
    res_v[pl.ds(16, 16)] = acc1
    pltpu.sync_copy(res_v, out_hbm.at[pl.ds(q0, _QPT)])


@functools.cache
def _get_select():
    @functools.partial(
        pl.kernel,
        out_type=jax.ShapeDtypeStruct((Q,), jnp.float32),
        mesh=plsc.VectorSubcoreMesh(core_axis_name="c", subcore_axis_name="s"),
        scratch_types=[
            pltpu.VMEM((NCHUNK, _QPT, 16), jnp.float32),
            pltpu.VMEM((_QPT, 10, GROUP), jnp.float32),
            pltpu.VMEM((_QPT,), jnp.float32),
            pltpu.SemaphoreType.DMA,
        ],
        compiler_params=pltpu.CompilerParams(
            needs_layout_passes=False, use_tc_tiling_on_sc=False
        ),
    )
    def _select(sg_hbm, gmax_hbm, out_hbm, gm_v, rows_v, res_v, sem):
        _select_body(sg_hbm, gmax_hbm, out_hbm, gm_v, rows_v, res_v, sem)

    return _select


def _scores(features, keys_p, interpret=False):
    return pl.pallas_call(
        _score_kernel,
        grid=(NCHUNK,),
        in_specs=[
            pl.BlockSpec((Q, D), lambda i: (0, 0)),
            pl.BlockSpec((CHUNK, D), lambda i: (i, 0)),
        ],
        out_specs=[
            pl.BlockSpec((GPC, Q, GROUP), lambda i: (i, 0, 0)),
            pl.BlockSpec((1, Q, GROUP), lambda i: (i, 0, 0)),
        ],
        out_shape=[
            jax.ShapeDtypeStruct((NGROUP, Q, GROUP), jnp.float32),
            jax.ShapeDtypeStruct((NCHUNK, Q, GROUP), jnp.float32),
        ],
        scratch_shapes=[pltpu.VMEM((Q, D), jnp.float32)],
        interpret=interpret,
    )(features, keys_p)


def kernel(features, logits, keys):
    del logits
    s3, gmax = _scores(features, keys)
    kth = _get_select()(s3, gmax)
    return kth.reshape(Q, 1)


# speedup vs baseline: 1.6546x; 1.0001x over previous
import functools

import jax
import jax.numpy as jnp
from jax import lax
from jax.experimental import pallas as pl
from jax.experimental.pallas import tpu as pltpu
from jax.experimental.pallas import tpu_sc as plsc

Q = 1024
D = 64
K = 100000
CHUNK = 2048
NCHUNK = 49
KPAD = NCHUNK * CHUNK
GROUP = 128
GPC = CHUNK // GROUP
NGROUP = NCHUNK * GPC
NEG = -3.0e38

_NTILE = 32
_QPT = Q // _NTILE


_JPART = (K % CHUNK) // GROUP


def _score_kernel(feat_ref, keys_ref, s_ref, gmax_ref, qn_scr):
    i = pl.program_id(0)

    @pl.when(i == 0)
    def _():
        f = feat_ref[...]
        qn = f * lax.rsqrt(jnp.sum(f * f, axis=1, keepdims=True))
        qn_scr[...] = qn.astype(jnp.bfloat16)

    qn = qn_scr[...]
    k = keys_ref[...]
    kn = (k * lax.rsqrt(jnp.maximum(jnp.sum(k * k, axis=1, keepdims=True),
                                    1e-30))).astype(jnp.bfloat16)
    last = i == NCHUNK - 1
    parts = []
    gparts = []
    for j in range(GPC):
        knj = kn[j * GROUP:(j + 1) * GROUP]
        sj = lax.dot_general(qn, knj, (((1,), (1,)), ((), ())),
                             preferred_element_type=jnp.float32)
        if j == _JPART:
            col = lax.broadcasted_iota(jnp.int32, (Q, GROUP), 1) \
                + i * CHUNK + j * GROUP
            sj = jnp.where(col < K, sj, NEG)
        gj = jnp.max(sj, axis=1, keepdims=True)
        if j > _JPART:
            gj = jnp.where(last, jnp.full((Q, 1), NEG, jnp.float32), gj)
        parts.append(sj)
        gparts.append(gj)
    s_ref[...] = jnp.stack(parts, axis=0)
    g16 = jnp.concatenate(gparts, axis=1)
    gmax_ref[0] = jnp.concatenate(
        [g16, jnp.full((Q, GROUP - GPC), NEG, jnp.float32)], axis=1)


def _merge16(run_asc, new_desc):
    return jnp.maximum(run_asc, new_desc)


def _select_body(sg_hbm, gmax_hbm, out_hbm, gm_v, rows_v, res_v, sem):
    c = lax.axis_index("c")
    s_ = lax.axis_index("s")
    wid = s_ * 2 + c
    q0 = wid * _QPT
    pltpu.sync_copy(gmax_hbm.at[:, pl.ds(q0, _QPT), pl.ds(0, GPC)], gm_v)

    _HQ = _QPT // 2

    def _p2_one(cc, g, rv, ri):
        ids = lax.iota(jnp.int32, 16) + cc * 16
        gs_n, gis = lax.sort((-g, ids), dimension=0, num_keys=1)
        gs = -gs_n
        take = rv >= gs
        mv = jnp.where(take, rv, gs)
        mi = jnp.where(take, ri, gis)
        return lax.sort((mv, mi), dimension=0, num_keys=1)

    def phase2(qi, _):
        def chunk_step(cc, carry):
            rva, ria, rvb, rib = carry
            rva, ria = _p2_one(cc, gm_v[cc, qi, :], rva, ria)
            rvb, rib = _p2_one(cc, gm_v[cc, qi + _HQ, :], rvb, rib)
            return (rva, ria, rvb, rib)

        rv0 = jnp.full((16,), NEG, jnp.float32)
        ri0 = jnp.zeros((16,), jnp.int32)
        rva, ria, rvb, rib = lax.fori_loop(
            0, NCHUNK, chunk_step, (rv0, ri0, rv0, ri0))
        for j in range(6, 16):
            pltpu.async_copy(sg_hbm.at[ria[j], q0 + qi],
                             rows_v.at[qi, j - 6], sem)
            pltpu.async_copy(sg_hbm.at[rib[j], q0 + qi + _HQ],
                             rows_v.at[qi + _HQ, j - 6], sem)
        return 0

    lax.fori_loop(0, _HQ, phase2, 0)

    def drain(qi, _):
        pltpu.make_async_copy(
            sg_hbm.at[0, pl.ds(0, 10), :], rows_v.at[qi], sem).wait()
        return 0

    lax.fori_loop(0, _QPT, drain, 0)

    def _p3_one(v, run2):
        vd = lax.rev(lax.sort(v, dimension=0), (0,))
        return lax.sort(_merge16(run2, vd), dimension=0)

    def phase3(qi, accs):
        def grp(gi, runs):
            def vstep(j, runs2):
                ra, rb = runs2
                ra = _p3_one(rows_v[qi, gi, pl.ds(j * 16, 16)], ra)
                rb = _p3_one(rows_v[qi + _HQ, gi, pl.ds(j * 16, 16)], rb)
                return (ra, rb)
            return lax.fori_loop(0, GROUP // 16, vstep, runs)

        run0 = jnp.full((16,), NEG, jnp.float32)
        ra, rb = lax.fori_loop(0, 10, grp, (run0, run0))
        acc0, acc1 = accs
        lane = lax.iota(jnp.int32, 16)
        acc0 = jnp.where(lane == qi, 2.0 - 2.0 * ra[6], acc0)
        acc1 = jnp.where(lane == qi, 2.0 - 2.0 * rb[6], acc1)
        return (acc0, acc1)

    z = jnp.zeros((16,), jnp.float32)
    acc0, acc1 = lax.fori_loop(0, _HQ, phase3, (z, z))
    res_v[pl.ds(0, 16)] = acc0
    res_v[pl.ds(16, 16)] = acc1
    pltpu.sync_copy(res_v, out_hbm.at[pl.ds(q0, _QPT)])


@functools.cache
def _get_select():
    @functools.partial(
        pl.kernel,
        out_type=jax.ShapeDtypeStruct((Q,), jnp.float32),
        mesh=plsc.VectorSubcoreMesh(core_axis_name="c", subcore_axis_name="s"),
        scratch_types=[
            pltpu.VMEM((NCHUNK, _QPT, 16), jnp.float32),
            pltpu.VMEM((_QPT, 10, GROUP), jnp.float32),
            pltpu.VMEM((_QPT,), jnp.float32),
            pltpu.SemaphoreType.DMA,
        ],
        compiler_params=pltpu.CompilerParams(
            needs_layout_passes=False, use_tc_tiling_on_sc=False
        ),
    )
    def _select(sg_hbm, gmax_hbm, out_hbm, gm_v, rows_v, res_v, sem):
        _select_body(sg_hbm, gmax_hbm, out_hbm, gm_v, rows_v, res_v, sem)

    return _select


def _scores(features, keys_p, interpret=False):
    return pl.pallas_call(
        _score_kernel,
        grid=(NCHUNK,),
        in_specs=[
            pl.BlockSpec((Q, D), lambda i: (0, 0)),
            pl.BlockSpec((CHUNK, D), lambda i: (i, 0)),
        ],
        out_specs=[
            pl.BlockSpec((GPC, Q, GROUP), lambda i: (i, 0, 0)),
            pl.BlockSpec((1, Q, GROUP), lambda i: (i, 0, 0)),
        ],
        out_shape=[
            jax.ShapeDtypeStruct((NGROUP, Q, GROUP), jnp.float32),
            jax.ShapeDtypeStruct((NCHUNK, Q, GROUP), jnp.float32),
        ],
        scratch_shapes=[pltpu.VMEM((Q, D), jnp.bfloat16)],
        interpret=interpret,
    )(features, keys_p)


def kernel(features, logits, keys):
    del logits
    s3, gmax = _scores(features, keys)
    kth = _get_select()(s3, gmax)
    return kth.reshape(Q, 1)
